# Initial kernel scaffold; baseline (speedup 1.0000x reference)
import jax
import jax.numpy as jnp
from jax.experimental import pallas as pl


def kernel(platform_id, industry_id, cta_id, platform_table, industry_table, cta_table, W1, b1, W2, b2):
    raise NotImplementedError("write your pallas kernel here")


# speedup vs baseline: 4.1438x; 4.1438x over previous
import jax
import jax.numpy as jnp
from jax.experimental import pallas as pl

_BLOCK = 2048


def _fused_kernel(pid_ref, iid_ref, cid_ref, tp_ref, ti_ref, tc_ref,
                  w1_ref, b1_ref, w2_ref, b2_ref, out_ref):
    blk = pid_ref.shape[0]
    vp, dp = tp_ref.shape
    vi, di = ti_ref.shape
    vc, dc = tc_ref.shape
    mp = jnp.dot(tp_ref[...], w1_ref[0:dp, :], preferred_element_type=jnp.float32)
    mi = jnp.dot(ti_ref[...], w1_ref[dp:dp + di, :], preferred_element_type=jnp.float32)
    mc = jnp.dot(tc_ref[...], w1_ref[dp + di:dp + di + dc, :], preferred_element_type=jnp.float32)
    oh_p = (pid_ref[...] == jax.lax.broadcasted_iota(jnp.int32, (blk, vp), 1)).astype(jnp.float32)
    oh_i = (iid_ref[...] == jax.lax.broadcasted_iota(jnp.int32, (blk, vi), 1)).astype(jnp.float32)
    oh_c = (cid_ref[...] == jax.lax.broadcasted_iota(jnp.int32, (blk, vc), 1)).astype(jnp.float32)
    h = (jnp.dot(oh_p, mp, preferred_element_type=jnp.float32)
         + jnp.dot(oh_i, mi, preferred_element_type=jnp.float32)
         + jnp.dot(oh_c, mc, preferred_element_type=jnp.float32)
         + b1_ref[...][None, :])
    h = jnp.maximum(h, 0.0)
    out_ref[...] = (jnp.dot(h, w2_ref[...], preferred_element_type=jnp.float32)
                    + b2_ref[...][None, :])


def kernel(platform_id, industry_id, cta_id, platform_table, industry_table,
           cta_table, W1, b1, W2, b2):
    B = platform_id.shape[0]
    blk = min(_BLOCK, B)
    grid = B // blk
    pid2 = platform_id.reshape(B, 1).astype(jnp.int32)
    iid2 = industry_id.reshape(B, 1).astype(jnp.int32)
    cid2 = cta_id.reshape(B, 1).astype(jnp.int32)
    d_out = W2.shape[1]
    return pl.pallas_call(
        _fused_kernel,
        grid=(grid,),
        in_specs=[
            pl.BlockSpec((blk, 1), lambda i: (i, 0)),
            pl.BlockSpec((blk, 1), lambda i: (i, 0)),
            pl.BlockSpec((blk, 1), lambda i: (i, 0)),
            pl.BlockSpec(platform_table.shape, lambda i: (0, 0)),
            pl.BlockSpec(industry_table.shape, lambda i: (0, 0)),
            pl.BlockSpec(cta_table.shape, lambda i: (0, 0)),
            pl.BlockSpec(W1.shape, lambda i: (0, 0)),
            pl.BlockSpec(b1.shape, lambda i: (0,)),
            pl.BlockSpec(W2.shape, lambda i: (0, 0)),
            pl.BlockSpec(b2.shape, lambda i: (0,)),
        ],
        out_specs=pl.BlockSpec((blk, d_out), lambda i: (i, 0)),
        out_shape=jax.ShapeDtypeStruct((B, d_out), jnp.float32),
    )(pid2, iid2, cid2, platform_table, industry_table, cta_table, W1, b1, W2, b2)


# speedup vs baseline: 7.7645x; 1.8738x over previous
import jax
import jax.numpy as jnp
from jax.experimental import pallas as pl

_BLOCK = 2048


def _fused_kernel(pid_ref, iid_ref, cid_ref, tp_ref, ti_ref, tc_ref,
                  w1_ref, b1_ref, w2_ref, b2_ref, out_ref):
    blk = pid_ref.shape[0]
    vp, dp = tp_ref.shape
    vi, di = ti_ref.shape
    vc, dc = tc_ref.shape
    mp = jnp.dot(tp_ref[...], w1_ref[0:dp, :], preferred_element_type=jnp.float32)
    mi = jnp.dot(ti_ref[...], w1_ref[dp:dp + di, :], preferred_element_type=jnp.float32)
    mc = jnp.dot(tc_ref[...], w1_ref[dp + di:dp + di + dc, :], preferred_element_type=jnp.float32)
    pid = pid_ref[...][:, None]
    iid = iid_ref[...][:, None]
    cid = cid_ref[...][:, None]
    oh_p = (pid == jax.lax.broadcasted_iota(jnp.int32, (blk, vp), 1)).astype(jnp.float32)
    oh_i = (iid == jax.lax.broadcasted_iota(jnp.int32, (blk, vi), 1)).astype(jnp.float32)
    oh_c = (cid == jax.lax.broadcasted_iota(jnp.int32, (blk, vc), 1)).astype(jnp.float32)
    h = (jnp.dot(oh_p, mp, preferred_element_type=jnp.float32)
         + jnp.dot(oh_i, mi, preferred_element_type=jnp.float32)
         + jnp.dot(oh_c, mc, preferred_element_type=jnp.float32)
         + b1_ref[...][None, :])
    h = jnp.maximum(h, 0.0)
    out_ref[...] = (jnp.dot(h, w2_ref[...], preferred_element_type=jnp.float32)
                    + b2_ref[...][None, :])


def kernel(platform_id, industry_id, cta_id, platform_table, industry_table,
           cta_table, W1, b1, W2, b2):
    B = platform_id.shape[0]
    blk = min(_BLOCK, B)
    grid = B // blk
    pid2 = platform_id.astype(jnp.int32)
    iid2 = industry_id.astype(jnp.int32)
    cid2 = cta_id.astype(jnp.int32)
    d_out = W2.shape[1]
    return pl.pallas_call(
        _fused_kernel,
        grid=(grid,),
        in_specs=[
            pl.BlockSpec((blk,), lambda i: (i,)),
            pl.BlockSpec((blk,), lambda i: (i,)),
            pl.BlockSpec((blk,), lambda i: (i,)),
            pl.BlockSpec(platform_table.shape, lambda i: (0, 0)),
            pl.BlockSpec(industry_table.shape, lambda i: (0, 0)),
            pl.BlockSpec(cta_table.shape, lambda i: (0, 0)),
            pl.BlockSpec(W1.shape, lambda i: (0, 0)),
            pl.BlockSpec(b1.shape, lambda i: (0,)),
            pl.BlockSpec(W2.shape, lambda i: (0, 0)),
            pl.BlockSpec(b2.shape, lambda i: (0,)),
        ],
        out_specs=pl.BlockSpec((blk, d_out), lambda i: (i, 0)),
        out_shape=jax.ShapeDtypeStruct((B, d_out), jnp.float32),
    )(pid2, iid2, cid2, platform_table, industry_table, cta_table, W1, b1, W2, b2)


# speedup vs baseline: 8.2283x; 1.0597x over previous
import jax
import jax.numpy as jnp
from jax.experimental import pallas as pl

_BLOCK = 4096


def _fused_kernel(pid_ref, iid_ref, cid_ref, tp_ref, ti_ref, tc_ref,
                  w1_ref, b1_ref, w2_ref, b2_ref, out_ref):
    blk = pid_ref.shape[0]
    vp, dp = tp_ref.shape
    vi, di = ti_ref.shape
    vc, dc = tc_ref.shape
    mp = jnp.dot(tp_ref[...], w1_ref[0:dp, :], preferred_element_type=jnp.float32)
    mi = jnp.dot(ti_ref[...], w1_ref[dp:dp + di, :], preferred_element_type=jnp.float32)
    mc = jnp.dot(tc_ref[...], w1_ref[dp + di:dp + di + dc, :], preferred_element_type=jnp.float32)
    pid = pid_ref[...][:, None]
    iid = iid_ref[...][:, None]
    cid = cid_ref[...][:, None]
    oh_p = (pid == jax.lax.broadcasted_iota(jnp.int32, (blk, vp), 1)).astype(jnp.float32)
    oh_i = (iid == jax.lax.broadcasted_iota(jnp.int32, (blk, vi), 1)).astype(jnp.float32)
    oh_c = (cid == jax.lax.broadcasted_iota(jnp.int32, (blk, vc), 1)).astype(jnp.float32)
    h = (jnp.dot(oh_p, mp, preferred_element_type=jnp.float32)
         + jnp.dot(oh_i, mi, preferred_element_type=jnp.float32)
         + jnp.dot(oh_c, mc, preferred_element_type=jnp.float32)
         + b1_ref[...][None, :])
    h = jnp.maximum(h, 0.0)
    out_ref[...] = (jnp.dot(h, w2_ref[...], preferred_element_type=jnp.float32)
                    + b2_ref[...][None, :])


def kernel(platform_id, industry_id, cta_id, platform_table, industry_table,
           cta_table, W1, b1, W2, b2):
    B = platform_id.shape[0]
    blk = min(_BLOCK, B)
    grid = B // blk
    pid2 = platform_id.astype(jnp.int32)
    iid2 = industry_id.astype(jnp.int32)
    cid2 = cta_id.astype(jnp.int32)
    d_out = W2.shape[1]
    return pl.pallas_call(
        _fused_kernel,
        grid=(grid,),
        in_specs=[
            pl.BlockSpec((blk,), lambda i: (i,)),
            pl.BlockSpec((blk,), lambda i: (i,)),
            pl.BlockSpec((blk,), lambda i: (i,)),
            pl.BlockSpec(platform_table.shape, lambda i: (0, 0)),
            pl.BlockSpec(industry_table.shape, lambda i: (0, 0)),
            pl.BlockSpec(cta_table.shape, lambda i: (0, 0)),
            pl.BlockSpec(W1.shape, lambda i: (0, 0)),
            pl.BlockSpec(b1.shape, lambda i: (0,)),
            pl.BlockSpec(W2.shape, lambda i: (0, 0)),
            pl.BlockSpec(b2.shape, lambda i: (0,)),
        ],
        out_specs=pl.BlockSpec((blk, d_out), lambda i: (i, 0)),
        out_shape=jax.ShapeDtypeStruct((B, d_out), jnp.float32),
    )(pid2, iid2, cid2, platform_table, industry_table, cta_table, W1, b1, W2, b2)


# speedup vs baseline: 8.2749x; 1.0057x over previous
import jax
import jax.numpy as jnp
from jax.experimental import pallas as pl

_BLOCK = 8192


def _fused_kernel(pid_ref, iid_ref, cid_ref, tp_ref, ti_ref, tc_ref,
                  w1_ref, b1_ref, w2_ref, b2_ref, out_ref):
    blk = pid_ref.shape[0]
    vp, dp = tp_ref.shape
    vi, di = ti_ref.shape
    vc, dc = tc_ref.shape
    mp = jnp.dot(tp_ref[...], w1_ref[0:dp, :], preferred_element_type=jnp.float32)
    mi = jnp.dot(ti_ref[...], w1_ref[dp:dp + di, :], preferred_element_type=jnp.float32)
    mc = jnp.dot(tc_ref[...], w1_ref[dp + di:dp + di + dc, :], preferred_element_type=jnp.float32)
    pid = pid_ref[...][:, None]
    iid = iid_ref[...][:, None]
    cid = cid_ref[...][:, None]
    oh_p = (pid == jax.lax.broadcasted_iota(jnp.int32, (blk, vp), 1)).astype(jnp.float32)
    oh_i = (iid == jax.lax.broadcasted_iota(jnp.int32, (blk, vi), 1)).astype(jnp.float32)
    oh_c = (cid == jax.lax.broadcasted_iota(jnp.int32, (blk, vc), 1)).astype(jnp.float32)
    h = (jnp.dot(oh_p, mp, preferred_element_type=jnp.float32)
         + jnp.dot(oh_i, mi, preferred_element_type=jnp.float32)
         + jnp.dot(oh_c, mc, preferred_element_type=jnp.float32)
         + b1_ref[...][None, :])
    h = jnp.maximum(h, 0.0)
    out_ref[...] = (jnp.dot(h, w2_ref[...], preferred_element_type=jnp.float32)
                    + b2_ref[...][None, :])


def kernel(platform_id, industry_id, cta_id, platform_table, industry_table,
           cta_table, W1, b1, W2, b2):
    B = platform_id.shape[0]
    blk = min(_BLOCK, B)
    grid = B // blk
    pid2 = platform_id.astype(jnp.int32)
    iid2 = industry_id.astype(jnp.int32)
    cid2 = cta_id.astype(jnp.int32)
    d_out = W2.shape[1]
    return pl.pallas_call(
        _fused_kernel,
        grid=(grid,),
        in_specs=[
            pl.BlockSpec((blk,), lambda i: (i,)),
            pl.BlockSpec((blk,), lambda i: (i,)),
            pl.BlockSpec((blk,), lambda i: (i,)),
            pl.BlockSpec(platform_table.shape, lambda i: (0, 0)),
            pl.BlockSpec(industry_table.shape, lambda i: (0, 0)),
            pl.BlockSpec(cta_table.shape, lambda i: (0, 0)),
            pl.BlockSpec(W1.shape, lambda i: (0, 0)),
            pl.BlockSpec(b1.shape, lambda i: (0,)),
            pl.BlockSpec(W2.shape, lambda i: (0, 0)),
            pl.BlockSpec(b2.shape, lambda i: (0,)),
        ],
        out_specs=pl.BlockSpec((blk, d_out), lambda i: (i, 0)),
        out_shape=jax.ShapeDtypeStruct((B, d_out), jnp.float32),
    )(pid2, iid2, cid2, platform_table, industry_table, cta_table, W1, b1, W2, b2)
